# baseline (speedup 1.0000x reference)
import numpy as np
import jax
import jax.numpy as jnp
from jax.experimental import pallas as pl

_ROWS, _COLS = 16384, 200


def _np_rotl(x, d):
    d = np.uint32(d)
    return ((x << d) | (x >> np.uint32(32 - d))).astype(np.uint32)


def _np_threefry2x32(ks0, ks1, x0, x1):
    with np.errstate(over="ignore"):
        ks2 = np.uint32(ks0 ^ ks1 ^ np.uint32(0x1BD11BDA))
        ks = (np.uint32(ks0), np.uint32(ks1), ks2)
        x0 = (x0 + ks[0]).astype(np.uint32)
        x1 = (x1 + ks[1]).astype(np.uint32)
        rots = ((13, 15, 26, 6), (17, 29, 16, 24))
        for i in range(5):
            for r in rots[i % 2]:
                x0 = (x0 + x1).astype(np.uint32)
                x1 = _np_rotl(x1, r)
                x1 = (x1 ^ x0).astype(np.uint32)
            x0 = (x0 + ks[(i + 1) % 3]).astype(np.uint32)
            x1 = (x1 + ks[(i + 2) % 3] + np.uint32(i + 1)).astype(np.uint32)
    return x0, x1


_s0, _s1 = _np_threefry2x32(np.uint32(0), np.uint32(42),
                            np.zeros(2, np.uint32), np.arange(2, dtype=np.uint32))
_MK0, _MK1 = int(_s0[0]), int(_s1[0])
_t0, _t1 = _np_threefry2x32(np.uint32(_s0[1]), np.uint32(_s1[1]),
                            np.zeros(2, np.uint32), np.arange(2, dtype=np.uint32))
_CK0, _CK1 = int(_t0[1]), int(_t1[1])

_MASK_THRESH = 838861


def _tf_hash(k0, k1, x1):
    k0 = jnp.uint32(k0)
    k1 = jnp.uint32(k1)
    k2 = jnp.uint32(k0 ^ k1 ^ 0x1BD11BDA)
    ks = (k0, k1, k2)
    x0 = jnp.full_like(x1, k0)
    x1 = x1 + k1
    rots = ((13, 15, 26, 6), (17, 29, 16, 24))
    for i in range(5):
        for r in rots[i % 2]:
            x0 = x0 + x1
            x1 = (x1 << r) | (x1 >> (32 - r))
            x1 = x1 ^ x0
        x0 = x0 + ks[(i + 1) % 3]
        x1 = x1 + ks[(i + 2) % 3] + jnp.uint32(i + 1)
    return x0 ^ x1


def _body(tok_ref, out_ref, *, block_rows):
    pid = pl.program_id(0)
    r = jax.lax.broadcasted_iota(jnp.int32, (block_rows, _COLS), 0)
    c = jax.lax.broadcasted_iota(jnp.int32, (block_rows, _COLS), 1)
    idx = ((pid * block_rows + r) * _COLS + c).astype(jnp.uint32)
    mbits = _tf_hash(_MK0, _MK1, idx)
    vbits = _tf_hash(_CK0, _CK1, idx)
    mask = (mbits >> 9) < jnp.uint32(_MASK_THRESH)
    repl = (vbits & jnp.uint32(3)).astype(jnp.int32)
    out_ref[...] = jnp.where(mask, repl, tok_ref[...])


def kernel(tokens):
    block_rows = 1024
    grid = _ROWS // block_rows
    import functools
    return pl.pallas_call(
        functools.partial(_body, block_rows=block_rows),
        grid=(grid,),
        in_specs=[pl.BlockSpec((block_rows, _COLS), lambda i: (i, 0))],
        out_specs=pl.BlockSpec((block_rows, _COLS), lambda i: (i, 0)),
        out_shape=jax.ShapeDtypeStruct((_ROWS, _COLS), jnp.int32),
    )(tokens)
